# baseline (speedup 1.0000x reference)
import functools
import math

import jax
import jax.numpy as jnp
from jax import lax
from jax.experimental import pallas as pl
from jax.experimental.pallas import tpu as pltpu
from jax.experimental.pallas import tpu_sc as plsc

NC = 2
NS = 16
NW = NC * NS
L = 16
NBUF = 4


@functools.partial(jax.jit, static_argnames=("n", "j", "d"))
def _embed(x, lut, *, n, j, d):
    ipw = n // NW
    ch = ipw // 128
    assert ch == 4
    ng = j * ch
    scale = jnp.float32(math.sqrt(d))

    mesh = plsc.VectorSubcoreMesh(
        core_axis_name="c", subcore_axis_name="s",
        num_cores=NC, num_subcores=NS)

    def body(x_hbm, lut_hbm, out_hbm, idx_scr, idx_t, ins, stages, sins, souts):
        wid = lax.axis_index("s") * NC + lax.axis_index("c")
        ibase = wid * ipw
        itbase = wid * ch
        pltpu.sync_copy(x_hbm.at[pl.ds(ibase, ipw)], idx_scr)

        iota = lax.iota(jnp.int32, L)
        @pl.loop(0, j)
        def _(jj):
            jv = jnp.full((L,), jj, jnp.int32)
            for m in range(ipw // L):
                v = plsc.load_gather(idx_scr, [iota + (m * L), jv])
                idx_t[jj, pl.ds(m * L, L)] = v

        koffs = [(iota + m * L) >> 3 for m in range(d // L)]
        kiffs = [(iota + m * L) & 7 for m in range(d // L)]
        zero = jnp.zeros((L,), jnp.int32)

        def start_gather(g, b):
            jg = g >> 2
            cg = g & 3
            pltpu.async_copy(
                lut_hbm.at[idx_t.at[jg, pl.ds(cg * 128, 128)]], ins[b], sins[b])

        def wait_gather(b):
            pltpu.make_async_copy(
                lut_hbm.at[idx_t.at[0, pl.ds(0, 128)]], ins[b], sins[b]).wait()

        def start_out(g, b):
            jg = g >> 2
            cg = g & 3
            pltpu.async_copy(
                stages[b],
                out_hbm.at[jg, :, pl.ds(itbase + cg, 1)], souts[b])

        def wait_out(b):
            pltpu.make_async_copy(
                stages[b], out_hbm.at[0, :, pl.ds(0, 1)], souts[b]).wait()

        def transform(b):
            src, dst = ins[b], stages[b]

            @plsc.parallel_loop(0, 128, unroll=2)
            def _(il):
                ilv = zero + il
                for m in range(d // L):
                    v = src[il, pl.ds(m * L, L)] * scale
                    plsc.store_scatter(dst, [koffs[m], zero, kiffs[m], ilv], v)

        def do_group(g, b, first, last):
            wait_gather(b)
            if not first:
                wait_out(b)
            transform(b)
            if not last:
                start_gather(g + NBUF, b)
            start_out(g, b)

        for b in range(NBUF):
            start_gather(b, b)
        for b in range(NBUF):
            do_group(b, b, True, False)
        nr = ng // NBUF

        @pl.loop(1, nr - 1)
        def _(rr):
            for b in range(NBUF):
                do_group(rr * NBUF + b, b, False, False)

        for b in range(NBUF):
            do_group((nr - 1) * NBUF + b, b, False, True)
        for b in range(NBUF):
            wait_out(b)

    f32 = jnp.float32
    run = pl.kernel(
        body,
        out_type=jax.ShapeDtypeStruct((j, d // 8, n // 128, 8, 128), f32),
        mesh=mesh,
        scratch_types=[
            pltpu.VMEM((ipw, j), jnp.int32),
            pltpu.VMEM((j, ipw), jnp.int32),
            tuple(pltpu.VMEM((128, d), f32) for _ in range(NBUF)),
            tuple(pltpu.VMEM((d // 8, 1, 8, 128), f32) for _ in range(NBUF)),
            tuple(pltpu.SemaphoreType.DMA for _ in range(NBUF)),
            tuple(pltpu.SemaphoreType.DMA for _ in range(NBUF)),
        ],
        compiler_params=pltpu.CompilerParams(use_tc_tiling_on_sc=False, needs_layout_passes=False),
    )
    return run(x, lut)


def kernel(x, lut):
    n, j = x.shape
    d = lut.shape[1]
    assert n % (NW * 128) == 0 and d % L == 0, (n, d)
    if x.dtype != jnp.int32:
        x = x.astype(jnp.int32)
    out5 = _embed(x, lut, n=n, j=j, d=d)
    return out5.transpose(2, 4, 0, 1, 3).reshape(n, j, d)
